# baseline (speedup 1.0000x reference)
import functools

import jax
import jax.numpy as jnp
from jax import lax
from jax.experimental import pallas as pl
from jax.experimental.pallas import tpu as pltpu
from jax.experimental.pallas import tpu_sc as plsc

_L = 16


def kernel(x, pos_table):
    B, S, D = x.shape
    NC, NS = 2, 16
    NW = NC * NS
    SW = S // NW
    R = 8
    NCH = SW // R

    mesh = plsc.VectorSubcoreMesh(core_axis_name="c", subcore_axis_name="s")

    @functools.partial(
        pl.kernel,
        out_type=jax.ShapeDtypeStruct((B, S, D), jnp.float32),
        mesh=mesh,
        scratch_types=[
            pltpu.VMEM((2, R, D), jnp.float32),
            pltpu.VMEM((3, B, R, D), jnp.float32),
            pltpu.SemaphoreType.DMA,
            pltpu.SemaphoreType.DMA,
            pltpu.SemaphoreType.DMA,
            pltpu.SemaphoreType.DMA,
            pltpu.SemaphoreType.DMA,
            pltpu.SemaphoreType.DMA,
            pltpu.SemaphoreType.DMA,
            pltpu.SemaphoreType.DMA,
        ],
        compiler_params=pltpu.CompilerParams(use_tc_tiling_on_sc=True),
    )
    def body(x_hbm, t_hbm, o_hbm, tbuf, xbuf,
             sin0, sin1, sin2, sout0, sout1, sout2, st0, st1):
        wid = lax.axis_index("s") * NC + lax.axis_index("c")
        s0 = wid * SW
        sin = (sin0, sin1, sin2)
        sout = (sout0, sout1, sout2)
        st = (st0, st1)

        def start_tin(c, tslot):
            tr = s0 + c * R
            return pltpu.async_copy(t_hbm.at[pl.ds(tr, R), :], tbuf.at[tslot], st[tslot])

        def start_in(c, slot):
            sr = s0 + c * R
            return pltpu.async_copy(x_hbm.at[:, pl.ds(sr, R), :], xbuf.at[slot], sin[slot])

        HD = D // 4

        def start_out(c, slot, part, nq):
            sr = s0 + c * R
            col = part * nq * HD
            return pltpu.async_copy(
                xbuf.at[slot, :, :, pl.ds(col, nq * HD)],
                o_hbm.at[:, pl.ds(sr, R), pl.ds(col, nq * HD)],
                sout[slot],
            )

        def wait_in(slot):
            pltpu.make_async_copy(
                x_hbm.at[:, pl.ds(0, R), :], xbuf.at[slot], sin[slot]
            ).wait()

        def wait_tin(tslot):
            pltpu.make_async_copy(t_hbm.at[pl.ds(0, R), :], tbuf.at[tslot], st[tslot]).wait()

        def wait_out(slot):
            for part in range(4):
                pltpu.make_async_copy(
                    xbuf.at[slot, :, :, pl.ds(0, HD)],
                    o_hbm.at[:, pl.ds(0, R), pl.ds(0, HD)],
                    sout[slot],
                ).wait()

        def compute(c, slot, tslot, j0, j1):
            def step(j, carry):
                off = j * _L
                for r in range(R):
                    t = tbuf[tslot, r, pl.ds(off, _L)]
                    for b in range(B):
                        plsc.addupdate(xbuf.at[slot, b, r, pl.ds(off, _L)], t)
                return carry

            lax.fori_loop(j0, j1, step, 0)

        def chunk_body(c, slot, tslot, first, last, nparts=2):
            nq = 4 // nparts
            hj = D // _L // nparts
            wait_in(slot)
            wait_tin(tslot)
            for part in range(nparts):
                compute(c, slot, tslot, part * hj, (part + 1) * hj)
                if part == nparts - 1 and not last:
                    start_tin(c + 2, tslot)
                start_out(c, slot, part, nq)
            if not first:
                wait_out((slot + 2) % 3)
                if not last:
                    start_in(c + 2, (slot + 2) % 3)

        start_in(0, 0)
        start_tin(0, 0)
        start_in(1, 1)
        start_tin(1, 1)
        start_in(2, 2)

        chunk_body(0, 0, 0, True, False)
        chunk_body(1, 1, 1, False, False)
        chunk_body(2, 2, 0, False, False)

        NROLL = (NCH - 3 - 3) // 6

        def outer(g, carry):
            base = 3 + g * 6
            for k in range(6):
                c = base + k
                chunk_body(c, (3 + k) % 3, (3 + k) % 2, False, False, nparts=4)
            return carry

        lax.fori_loop(0, NROLL, outer, 0)

        for c in range(3 + NROLL * 6, NCH):
            chunk_body(c, c % 3, c % 2, False, c + 2 >= NCH)
        wait_out((NCH - 1) % 3)

    return body(x, pos_table)


# speedup vs baseline: 1.0188x; 1.0188x over previous
import functools

import jax
import jax.numpy as jnp
from jax import lax
from jax.experimental import pallas as pl
from jax.experimental.pallas import tpu as pltpu
from jax.experimental.pallas import tpu_sc as plsc

_L = 16


def kernel(x, pos_table):
    B, S, D = x.shape
    NC, NS = 2, 16
    NW = NC * NS
    SW = S // NW
    R = 8
    NCH = SW // R

    mesh = plsc.VectorSubcoreMesh(core_axis_name="c", subcore_axis_name="s")

    @functools.partial(
        pl.kernel,
        out_type=jax.ShapeDtypeStruct((B, S, D), jnp.float32),
        mesh=mesh,
        scratch_types=[
            pltpu.VMEM((2, R, D), jnp.float32),
            pltpu.VMEM((3, B, R, D), jnp.float32),
            pltpu.SemaphoreType.DMA,
            pltpu.SemaphoreType.DMA,
            pltpu.SemaphoreType.DMA,
            pltpu.SemaphoreType.DMA,
            pltpu.SemaphoreType.DMA,
            pltpu.SemaphoreType.DMA,
            pltpu.SemaphoreType.DMA,
            pltpu.SemaphoreType.DMA,
        ],
        compiler_params=pltpu.CompilerParams(use_tc_tiling_on_sc=True),
    )
    def body(x_hbm, t_hbm, o_hbm, tbuf, xbuf,
             sin0, sin1, sin2, sout0, sout1, sout2, st0, st1):
        wid = lax.axis_index("s") * NC + lax.axis_index("c")
        s0 = wid * SW
        sin = (sin0, sin1, sin2)
        sout = (sout0, sout1, sout2)
        st = (st0, st1)

        def start_tin(c, tslot):
            tr = s0 + c * R
            return pltpu.async_copy(t_hbm.at[pl.ds(tr, R), :], tbuf.at[tslot], st[tslot])

        def start_in(c, slot):
            sr = s0 + c * R
            return pltpu.async_copy(x_hbm.at[:, pl.ds(sr, R), :], xbuf.at[slot], sin[slot])

        HD = D // 4

        def start_out(c, slot, part, nq):
            sr = s0 + c * R
            col = part * nq * HD
            return pltpu.async_copy(
                xbuf.at[slot, :, :, pl.ds(col, nq * HD)],
                o_hbm.at[:, pl.ds(sr, R), pl.ds(col, nq * HD)],
                sout[slot],
            )

        def wait_in(slot):
            pltpu.make_async_copy(
                x_hbm.at[:, pl.ds(0, R), :], xbuf.at[slot], sin[slot]
            ).wait()

        def wait_tin(tslot):
            pltpu.make_async_copy(t_hbm.at[pl.ds(0, R), :], tbuf.at[tslot], st[tslot]).wait()

        def wait_out(slot):
            for part in range(4):
                pltpu.make_async_copy(
                    xbuf.at[slot, :, :, pl.ds(0, HD)],
                    o_hbm.at[:, pl.ds(0, R), pl.ds(0, HD)],
                    sout[slot],
                ).wait()

        def compute(c, slot, tslot, j0, j1):
            def step(j, carry):
                off = j * _L
                for r in range(R):
                    t = tbuf[tslot, r, pl.ds(off, _L)]
                    for b in range(B):
                        plsc.addupdate(xbuf.at[slot, b, r, pl.ds(off, _L)], t)
                return carry

            lax.fori_loop(j0, j1, step, 0)

        def chunk_body(c, slot, tslot, first, last, nparts=2):
            nq = 4 // nparts
            hj = D // _L // nparts
            wait_in(slot)
            wait_tin(tslot)
            for part in range(nparts):
                compute(c, slot, tslot, part * hj, (part + 1) * hj)
                if part == nparts - 1 and not last:
                    start_tin(c + 2, tslot)
                start_out(c, slot, part, nq)
            if not first:
                wait_out((slot + 2) % 3)
                if not last:
                    start_in(c + 2, (slot + 2) % 3)

        start_in(0, 0)
        start_tin(0, 0)
        start_in(1, 1)
        start_tin(1, 1)
        start_in(2, 2)

        chunk_body(0, 0, 0, True, False)
        chunk_body(1, 1, 1, False, False)
        chunk_body(2, 2, 0, False, False)

        NROLL = (NCH - 3 - 3) // 6

        def outer(g, carry):
            base = 3 + g * 6
            for k in range(6):
                c = base + k
                chunk_body(c, (3 + k) % 3, (3 + k) % 2, False, False)
            return carry

        lax.fori_loop(0, NROLL, outer, 0)

        for c in range(3 + NROLL * 6, NCH):
            chunk_body(c, c % 3, c % 2, False, c + 2 >= NCH)
        wait_out((NCH - 1) % 3)

    return body(x, pos_table)
